# Initial kernel scaffold; baseline (speedup 1.0000x reference)
import jax
import jax.numpy as jnp
from jax.experimental import pallas as pl


def kernel(x, edge_index, batch, W1, b1, W2, b2, W3, b3, g1, be1, g2, be2, g3, be3, fW1, fb1, fW2, fb2, fW3, fb3):
    raise NotImplementedError("write your pallas kernel here")


# speedup vs baseline: 6.7359x; 6.7359x over previous
import functools

import jax
import jax.numpy as jnp
from jax import lax
from jax.experimental import pallas as pl
from jax.experimental.pallas import tpu as pltpu
from jax.experimental.pallas import tpu_sc as plsc

_N = 10000
_E = 320000
_D = 128
_G = 64
_NC = 2
_NS = 16
_NW = _NC * _NS
_NPAD = 10112
_RPTC = _NPAD // _NS
_K = 98
_C = 104
_EPAD = _NW * _K * _C

_HIGH = lax.Precision.HIGHEST

def _sc_aggregate_body(h2_hbm, src_hbm, dst_hbm, zeros_hbm, out_hbm,
                       acc, sidx0, sidx1, didx0, didx1, rows0, rows1, sem0, sem1):
    c = lax.axis_index("c")
    s = lax.axis_index("s")
    tid = c * _NS + s
    base = s * _RPTC
    pltpu.sync_copy(zeros_hbm, acc.at[pl.ds(base, _RPTC)])
    plsc.subcore_barrier()

    pltpu.sync_copy(src_hbm.at[tid, 0], sidx0)
    pltpu.sync_copy(dst_hbm.at[tid, 0], didx0)
    pltpu.async_copy(h2_hbm.at[sidx0], rows0, sem0)

    def body(i, carry):
        j = 2 * i
        pltpu.sync_copy(src_hbm.at[tid, j + 1], sidx1)
        pltpu.sync_copy(dst_hbm.at[tid, j + 1], didx1)
        pltpu.make_async_copy(h2_hbm.at[sidx0], rows0, sem0).wait()
        pltpu.async_copy(h2_hbm.at[sidx1], rows1, sem1)
        pltpu.sync_copy(rows0, acc.at[didx0], add=True)

        @pl.when(i + 1 < _K // 2)
        def _():
            pltpu.sync_copy(src_hbm.at[tid, j + 2], sidx0)
            pltpu.sync_copy(dst_hbm.at[tid, j + 2], didx0)

        pltpu.make_async_copy(h2_hbm.at[sidx1], rows1, sem1).wait()

        @pl.when(i + 1 < _K // 2)
        def _():
            pltpu.async_copy(h2_hbm.at[sidx0], rows0, sem0)

        pltpu.sync_copy(rows1, acc.at[didx1], add=True)
        return carry

    lax.fori_loop(0, _K // 2, body, 0)
    plsc.subcore_barrier()
    pltpu.sync_copy(acc.at[pl.ds(base, _RPTC)], out_hbm.at[c, pl.ds(base, _RPTC)])


@functools.lru_cache(maxsize=None)
def _sc_kernels():
    mesh = plsc.VectorSubcoreMesh(core_axis_name="c", subcore_axis_name="s",
                                  num_cores=_NC, num_subcores=_NS)
    agg = pl.kernel(
        _sc_aggregate_body,
        out_type=jax.ShapeDtypeStruct((_NC, _NPAD, _D), jnp.float32),
        mesh=mesh,
        scratch_types=[
            pltpu.VMEM_SHARED((_NPAD, _D), jnp.float32),
            pltpu.VMEM((_C,), jnp.int32),
            pltpu.VMEM((_C,), jnp.int32),
            pltpu.VMEM((_C,), jnp.int32),
            pltpu.VMEM((_C,), jnp.int32),
            pltpu.VMEM((_C, _D), jnp.float32),
            pltpu.VMEM((_C, _D), jnp.float32),
            pltpu.SemaphoreType.DMA,
            pltpu.SemaphoreType.DMA,
        ],
    )
    return agg


def _ln_relu(st, g, be):
    h = jnp.maximum(st, 0.0)
    m = jnp.mean(h, axis=-1, keepdims=True)
    d = h - m
    v = jnp.mean(d * d, axis=-1, keepdims=True)
    return d * lax.rsqrt(v + 1e-5) * g + be


def _dot(a, b):
    return jnp.dot(a, b, preferred_element_type=jnp.float32, precision=_HIGH)


def _tc_first_body(deg_ref, x_ref, w_ref, dis_ref, h2_ref):
    deg = deg_ref[0, :, 0:1] + deg_ref[1, :, 0:1] + 1.0
    dis = lax.rsqrt(deg)
    dis_ref[...] = dis
    h2_ref[...] = _dot(x_ref[...], w_ref[...]) * dis


_tc_first = pl.pallas_call(
    _tc_first_body,
    out_shape=(jax.ShapeDtypeStruct((_NPAD, 1), jnp.float32),
               jax.ShapeDtypeStruct((_NPAD, _D), jnp.float32)),
)


def _tc_mid_body(p_ref, h2_ref, dis_ref, b_ref, g_ref, be_ref, w_ref, out_ref):
    dis = dis_ref[...]
    st = (p_ref[0] + p_ref[1] + h2_ref[...]) * dis + b_ref[...]
    xn = _ln_relu(st, g_ref[...], be_ref[...])
    out_ref[...] = _dot(xn, w_ref[...]) * dis


_tc_mid = pl.pallas_call(
    _tc_mid_body,
    out_shape=jax.ShapeDtypeStruct((_NPAD, _D), jnp.float32),
)


def _tc_final_body(p_ref, h2_ref, dis_ref, b_ref, g_ref, be_ref, batch_ref,
                   fw1_ref, fb1_ref, fw2_ref, fb2_ref, fw3_ref, fb3_ref, out_ref,
                   smax_ref):
    dis = dis_ref[...]
    st = (p_ref[0] + p_ref[1] + h2_ref[...]) * dis + b_ref[...]
    h = _ln_relu(st, g_ref[...], be_ref[...])
    bt = batch_ref[...]
    groups = lax.broadcasted_iota(jnp.int32, (_NPAD, _G), 1)
    onehot = (bt == groups).astype(jnp.float32)
    dn = (((0,), (0,)), ((), ()))
    ssum = lax.dot_general(onehot, h, dn,
                           precision=_HIGH, preferred_element_type=jnp.float32)
    ones_col = jnp.ones((_NPAD, 1), jnp.float32)
    cnt = lax.dot_general(onehot, ones_col, dn,
                          precision=_HIGH, preferred_element_type=jnp.float32)
    smean = ssum / jnp.maximum(cnt, 1.0)
    neg = jnp.float32(-jnp.inf)

    def gbody(g, carry):
        m = jnp.max(jnp.where(bt == g, h, neg), axis=0, keepdims=True)
        smax_ref[pl.ds(g, 1), :] = m
        return carry

    lax.fori_loop(0, _G, gbody, 0)
    smax = jnp.where(cnt > 0.0, smax_ref[...], 0.0)
    z = jnp.concatenate([smean, ssum, smax], axis=1)
    z = jnp.maximum(_dot(z, fw1_ref[...]) + fb1_ref[...], 0.0)
    z = jnp.maximum(_dot(z, fw2_ref[...]) + fb2_ref[...], 0.0)
    out_ref[...] = _dot(z, fw3_ref[...]) + fb3_ref[...]


_tc_final = pl.pallas_call(
    _tc_final_body,
    out_shape=jax.ShapeDtypeStruct((_G, 64), jnp.float32),
    scratch_shapes=[pltpu.VMEM((_G, _D), jnp.float32)],
)


def kernel(x, edge_index, batch, W1, b1, W2, b2, W3, b3, g1, be1, g2, be2,
           g3, be3, fW1, fb1, fW2, fb2, fW3, fb3):
    f32 = jnp.float32
    xpad = jnp.pad(x, ((0, _NPAD - _N), (0, 0)))
    epad = jnp.full((_EPAD - _E,), _NPAD - 1, jnp.int32)
    src3 = jnp.concatenate([edge_index[0], epad]).reshape(_NW, _K, _C)
    dst3 = jnp.concatenate([edge_index[1], epad]).reshape(_NW, _K, _C)
    ones_mat = jnp.ones((_NPAD, _D), f32)
    zeros128 = jnp.zeros((_RPTC, _D), f32)
    batch2d = jnp.pad(batch, (0, _NPAD - _N),
                      constant_values=_G).reshape(_NPAD, 1)
    r = lambda v: v.reshape(1, -1)

    _sc_aggregate = _sc_kernels()
    degp = _sc_aggregate(ones_mat, src3, dst3, zeros128)
    dis, h2 = _tc_first(degp, xpad, W1)
    p = _sc_aggregate(h2, src3, dst3, zeros128)
    h2 = _tc_mid(p, h2, dis, r(b1), r(g1), r(be1), W2)
    p = _sc_aggregate(h2, src3, dst3, zeros128)
    h2 = _tc_mid(p, h2, dis, r(b2), r(g2), r(be2), W3)
    p = _sc_aggregate(h2, src3, dst3, zeros128)
    z = _tc_final(p, h2, dis, r(b3), r(g3), r(be3), batch2d,
                  fW1, r(fb1), fW2, r(fb2), fW3, r(fb3))
    return z
